# baseline (speedup 1.0000x reference)
import jax
import jax.numpy as jnp
from jax.experimental import pallas as pl
from jax.experimental.pallas import tpu as pltpu


def _concat_body(r_ref, g_ref, o_ref, sem_r, sem_g):
    n = r_ref.shape[1]
    cp_r = pltpu.make_async_copy(r_ref, o_ref.at[:, :n, :], sem_r)
    cp_g = pltpu.make_async_copy(g_ref, o_ref.at[:, n:, :], sem_g)
    cp_r.start()
    cp_g.start()
    cp_r.wait()
    cp_g.wait()


def kernel(rois, gt_boxes):
    B, N, C = rois.shape
    _, G, _ = gt_boxes.shape
    return pl.pallas_call(
        _concat_body,
        in_specs=[
            pl.BlockSpec(memory_space=pl.ANY),
            pl.BlockSpec(memory_space=pl.ANY),
        ],
        out_specs=pl.BlockSpec(memory_space=pl.ANY),
        out_shape=jax.ShapeDtypeStruct((B, N + G, C), rois.dtype),
        scratch_shapes=[pltpu.SemaphoreType.DMA, pltpu.SemaphoreType.DMA],
    )(rois, gt_boxes)


# speedup vs baseline: 312.3934x; 312.3934x over previous
import jax
import jax.numpy as jnp
from jax.experimental import pallas as pl
from jax.experimental.pallas import tpu as pltpu


def _concat_body(r_ref, g_ref, o_ref):
    n = r_ref.shape[2]
    o_ref[:, :, :n] = r_ref[...]
    o_ref[:, :, n:] = g_ref[...]


def kernel(rois, gt_boxes):
    B, N, C = rois.shape
    _, G, _ = gt_boxes.shape
    r_t = jnp.transpose(rois, (0, 2, 1))
    g_t = jnp.transpose(gt_boxes, (0, 2, 1))
    out_t = pl.pallas_call(
        _concat_body,
        out_shape=jax.ShapeDtypeStruct((B, C, N + G), rois.dtype),
    )(r_t, g_t)
    return jnp.transpose(out_t, (0, 2, 1))
